# baseline (speedup 1.0000x reference)
import jax
import jax.numpy as jnp
from jax import lax
from jax.experimental import pallas as pl
from jax.experimental.pallas import tpu as pltpu
from jax.experimental.pallas import tpu_sc as plsc

N = 32768
NB = 16
NS = 16
CHUNK = N // NS
ITERS = CHUNK // 16
QA = 5
QP = 6
ACC = QA * NB * 16
PART = QP * 16


def _body(sl_hbm, oi_hbm, lab_hbm, foi_hbm, out_hbm,
          c0_v, c1_v, c2_v, oi_v, lab_v, foi_v, acc_v, tot_v, mrg_v, out_v,
          shared, sem):
    sid = lax.axis_index("s")
    base = sid * CHUNK
    sl = pl.ds(base, CHUNK)
    copies = [
        pltpu.async_copy(sl_hbm.at[pl.ds(base, CHUNK)], c0_v, sem),
        pltpu.async_copy(sl_hbm.at[pl.ds(N + base, CHUNK)], c1_v, sem),
        pltpu.async_copy(sl_hbm.at[pl.ds(2 * N + base, CHUNK)], c2_v, sem),
        pltpu.async_copy(oi_hbm.at[sl], oi_v, sem),
        pltpu.async_copy(lab_hbm.at[sl], lab_v, sem),
        pltpu.async_copy(foi_hbm.at[sl], foi_v, sem),
    ]

    iota = lax.iota(jnp.int32, 16)
    zero = jnp.zeros((16,), jnp.float32)
    ones = jnp.ones((16,), jnp.float32)

    @plsc.parallel_loop(0, ACC // 16, 1, unroll=4)
    def _(k):
        acc_v[pl.ds(k * 16, 16)] = zero

    for c in copies:
        c.wait()

    def block(off):
        c0 = c0_v[pl.ds(off, 16)]
        c1 = c1_v[pl.ds(off, 16)]
        c2 = c2_v[pl.ds(off, 16)]
        oi = oi_v[pl.ds(off, 16)]
        m = jnp.maximum(c0, jnp.maximum(c1, c2))
        sidx = oi * 16 + iota
        plsc.addupdate_scatter(acc_v, [sidx], ones)
        plsc.addupdate_scatter(acc_v, [sidx + 256], m)
        plsc.addupdate_scatter(acc_v, [sidx + 512], c0)
        plsc.addupdate_scatter(acc_v, [sidx + 768], c1 + c2)
        lab = lab_v[pl.ds(off, 16)]
        foi = foi_v[pl.ds(off, 16)]
        fidx = foi * 16 + iota
        p = (jnp.where(lab == 4, 1.0, 0.0)
             + jnp.where(lab == 1, 4096.0, 0.0)).astype(jnp.float32)
        plsc.addupdate_scatter(acc_v, [fidx + 1024], p)

    @plsc.parallel_loop(0, ITERS, 1, unroll=8)
    def _(i):
        block(i * 16)

    @plsc.parallel_loop(0, 16, 1, unroll=4, carry=(zero,) * QA)
    def tots(j, carry):
        rem = (iota + j) & 15
        return tuple(
            carry[q] + plsc.load_gather(acc_v, [q * 256 + iota * 16 + rem])
            for q in range(QA)
        )
    for q in range(QA - 1):
        tot_v[pl.ds(q * 16, 16)] = tots[q]
    pki = tots[4].astype(jnp.int32)
    tot_v[pl.ds(4 * 16, 16)] = (pki & 4095).astype(jnp.float32)
    tot_v[pl.ds(5 * 16, 16)] = (pki >> 12).astype(jnp.float32)

    pltpu.sync_copy(tot_v, shared.at[pl.ds(sid * PART, PART)])
    plsc.subcore_barrier()

    @pl.when(sid == 0)
    def _():
        pltpu.sync_copy(shared, mrg_v)

        def mrg(t, carry):
            b = t * PART
            return tuple(
                carry[q] + mrg_v[pl.ds(b + q * 16, 16)] for q in range(QP)
            )
        cnt, smax, s0, s12, c4, c1n = lax.fori_loop(
            0, NS, mrg, (zero,) * QP)
        avg = smax / cnt
        small = cnt < 6.0
        c4 = jnp.where(small, c4, 0.0)
        c1n = jnp.where(small, c1n, 0.0)
        x0 = s0 + c1n * avg - 5.0 * avg
        x1 = s12 + c4 * avg - avg
        j0 = 1.0 / (1.0 + jnp.exp(-x0))
        j1 = 1.0 / (1.0 + jnp.exp(-x1))
        zeros_i = jnp.zeros((16,), jnp.int32)
        plsc.store_scatter(out_v, [iota, zeros_i], j0)
        plsc.store_scatter(out_v, [iota, zeros_i + 1], j1)
        pltpu.sync_copy(out_v, out_hbm)


@jax.jit
def _run(sl_cols, oi, lab, foi):
    mesh = plsc.VectorSubcoreMesh(core_axis_name="c", subcore_axis_name="s",
                                  num_cores=1)
    f = pl.kernel(
        _body,
        out_type=jax.ShapeDtypeStruct((NB, 2), jnp.float32),
        mesh=mesh,
        compiler_params=pltpu.CompilerParams(
            use_tc_tiling_on_sc=False, needs_layout_passes=False),
        scratch_types=[
            pltpu.VMEM((CHUNK,), jnp.float32),
            pltpu.VMEM((CHUNK,), jnp.float32),
            pltpu.VMEM((CHUNK,), jnp.float32),
            pltpu.VMEM((CHUNK,), jnp.int32),
            pltpu.VMEM((CHUNK,), jnp.int32),
            pltpu.VMEM((CHUNK,), jnp.int32),
            pltpu.VMEM((ACC,), jnp.float32),
            pltpu.VMEM((PART,), jnp.float32),
            pltpu.VMEM((NS * PART,), jnp.float32),
            pltpu.VMEM((NB, 2), jnp.float32),
            pltpu.VMEM_SHARED((NS * PART,), jnp.float32),
            pltpu.SemaphoreType.DMA,
        ],
    )
    return f(sl_cols, oi, lab, foi)


def kernel(sub_logits, original_indices, full_sub_labels, full_original_indices):
    sl_flat = lax.reshape(sub_logits, (3 * N,), dimensions=(1, 0))
    oi = original_indices.astype(jnp.int32)
    lab = full_sub_labels.astype(jnp.int32)
    foi = full_original_indices.astype(jnp.int32)
    return _run(sl_flat, oi, lab, foi)


# speedup vs baseline: 1.0117x; 1.0117x over previous
import jax
import jax.numpy as jnp
from jax import lax
from jax.experimental import pallas as pl
from jax.experimental.pallas import tpu as pltpu
from jax.experimental.pallas import tpu_sc as plsc

N = 32768
NB = 16
NS = 16
CHUNK = N // NS
ITERS = CHUNK // 16
QA = 5
QP = 6
ACC = QA * NB * 16
PART = QP * 16


def _body(sl_hbm, oi_hbm, lab_hbm, foi_hbm, out_hbm,
          c0_v, c1_v, c2_v, oi_v, lab_v, foi_v, acc_v, tot_v, mrg_v, out_v,
          shared, sem):
    sid = lax.axis_index("s")
    base = sid * CHUNK
    sl = pl.ds(base, CHUNK)
    copies = [
        pltpu.async_copy(sl_hbm.at[pl.ds(base, CHUNK)], c0_v, sem),
        pltpu.async_copy(sl_hbm.at[pl.ds(N + base, CHUNK)], c1_v, sem),
        pltpu.async_copy(sl_hbm.at[pl.ds(2 * N + base, CHUNK)], c2_v, sem),
        pltpu.async_copy(oi_hbm.at[sl], oi_v, sem),
        pltpu.async_copy(lab_hbm.at[sl], lab_v, sem),
        pltpu.async_copy(foi_hbm.at[sl], foi_v, sem),
    ]

    iota = lax.iota(jnp.int32, 16)
    zero = jnp.zeros((16,), jnp.float32)
    ones = jnp.ones((16,), jnp.float32)

    @plsc.parallel_loop(0, ACC // 16, 1, unroll=4)
    def _(k):
        acc_v[pl.ds(k * 16, 16)] = zero

    for c in copies:
        c.wait()

    def block(off):
        c0 = c0_v[pl.ds(off, 16)]
        c1 = c1_v[pl.ds(off, 16)]
        c2 = c2_v[pl.ds(off, 16)]
        oi = oi_v[pl.ds(off, 16)]
        m = jnp.maximum(c0, jnp.maximum(c1, c2))
        sidx = oi * 16 + iota
        plsc.addupdate_scatter(acc_v, [sidx], ones)
        plsc.addupdate_scatter(acc_v, [sidx + 256], m)
        plsc.addupdate_scatter(acc_v, [sidx + 512], c0)
        plsc.addupdate_scatter(acc_v, [sidx + 768], c1 + c2)
        lab = lab_v[pl.ds(off, 16)]
        foi = foi_v[pl.ds(off, 16)]
        fidx = foi * 16 + iota
        p = (jnp.where(lab == 4, 1.0, 0.0)
             + jnp.where(lab == 1, 4096.0, 0.0)).astype(jnp.float32)
        plsc.addupdate_scatter(acc_v, [fidx + 1024], p)

    @plsc.parallel_loop(0, ITERS, 1, unroll=4)
    def _(i):
        block(i * 16)

    @plsc.parallel_loop(0, 16, 1, unroll=4, carry=(zero,) * QA)
    def tots(j, carry):
        rem = (iota + j) & 15
        return tuple(
            carry[q] + plsc.load_gather(acc_v, [q * 256 + iota * 16 + rem])
            for q in range(QA)
        )
    for q in range(QA - 1):
        tot_v[pl.ds(q * 16, 16)] = tots[q]
    pki = tots[4].astype(jnp.int32)
    tot_v[pl.ds(4 * 16, 16)] = (pki & 4095).astype(jnp.float32)
    tot_v[pl.ds(5 * 16, 16)] = (pki >> 12).astype(jnp.float32)

    pltpu.sync_copy(tot_v, shared.at[pl.ds(sid * PART, PART)])
    plsc.subcore_barrier()

    @pl.when(sid == 0)
    def _():
        pltpu.sync_copy(shared, mrg_v)

        def mrg(t, carry):
            b = t * PART
            return tuple(
                carry[q] + mrg_v[pl.ds(b + q * 16, 16)] for q in range(QP)
            )
        cnt, smax, s0, s12, c4, c1n = lax.fori_loop(
            0, NS, mrg, (zero,) * QP)
        avg = smax / cnt
        small = cnt < 6.0
        c4 = jnp.where(small, c4, 0.0)
        c1n = jnp.where(small, c1n, 0.0)
        x0 = s0 + c1n * avg - 5.0 * avg
        x1 = s12 + c4 * avg - avg
        j0 = 1.0 / (1.0 + jnp.exp(-x0))
        j1 = 1.0 / (1.0 + jnp.exp(-x1))
        plsc.store_scatter(out_v, [iota * 2], j0)
        plsc.store_scatter(out_v, [iota * 2 + 1], j1)
        pltpu.sync_copy(out_v, out_hbm)


@jax.jit
def _run(sl_cols, oi, lab, foi):
    mesh = plsc.VectorSubcoreMesh(core_axis_name="c", subcore_axis_name="s",
                                  num_cores=1)
    f = pl.kernel(
        _body,
        out_type=jax.ShapeDtypeStruct((2 * NB,), jnp.float32),
        mesh=mesh,
        compiler_params=pltpu.CompilerParams(
            use_tc_tiling_on_sc=False, needs_layout_passes=False),
        scratch_types=[
            pltpu.VMEM((CHUNK,), jnp.float32),
            pltpu.VMEM((CHUNK,), jnp.float32),
            pltpu.VMEM((CHUNK,), jnp.float32),
            pltpu.VMEM((CHUNK,), jnp.int32),
            pltpu.VMEM((CHUNK,), jnp.int32),
            pltpu.VMEM((CHUNK,), jnp.int32),
            pltpu.VMEM((ACC,), jnp.float32),
            pltpu.VMEM((PART,), jnp.float32),
            pltpu.VMEM((NS * PART,), jnp.float32),
            pltpu.VMEM((2 * NB,), jnp.float32),
            pltpu.VMEM_SHARED((NS * PART,), jnp.float32),
            pltpu.SemaphoreType.DMA,
        ],
    )
    return f(sl_cols, oi, lab, foi).reshape(NB, 2)


def kernel(sub_logits, original_indices, full_sub_labels, full_original_indices):
    sl_flat = lax.reshape(sub_logits, (3 * N,), dimensions=(1, 0))
    oi = original_indices.astype(jnp.int32)
    lab = full_sub_labels.astype(jnp.int32)
    foi = full_original_indices.astype(jnp.int32)
    return _run(sl_flat, oi, lab, foi)
